# baseline (speedup 1.0000x reference)
import functools

import jax
import jax.numpy as jnp
from jax import lax
from jax.experimental import pallas as pl
from jax.experimental.pallas import tpu as pltpu
from jax.experimental.pallas import tpu_sc as plsc

_BLK_K = 128
_A_BLOCKS = 4
_RBLK = 32
_NBUF = 6


def _merge(minv, argv, bmin, barg):
    better = bmin < minv
    return jnp.where(better, bmin, minv), jnp.where(better, barg, argv)


def _chunk_minarg(x, x2, c, base):
    dot = lax.dot_general(x, c, (((1,), (1,)), ((), ())),
                          preferred_element_type=jnp.float32)
    c2 = jnp.sum(c * c, axis=1)[None, :]
    d2 = jnp.maximum(x2 + c2 - 2.0 * dot, 0.0)
    bmin = jnp.min(d2, axis=1, keepdims=True)
    col = lax.broadcasted_iota(jnp.int32, d2.shape, 1)
    barg = (jnp.min(jnp.where(d2 == bmin, col, d2.shape[1]),
                    axis=1, keepdims=True) + base)
    return bmin, barg


def _phase_a_body(x_ref, c_ref, min_ref, arg_ref, min_scr, arg_scr):
    kb = pl.program_id(0)
    x = x_ref[...]
    x2 = jnp.sum(x * x, axis=1, keepdims=True)
    bmin, barg = _chunk_minarg(x, x2, c_ref[...], kb * _BLK_K)

    @pl.when(kb == 0)
    def _():
        min_scr[...] = bmin
        arg_scr[...] = barg

    @pl.when(kb > 0)
    def _():
        minv, argv = _merge(min_scr[...], arg_scr[...], bmin, barg)
        min_scr[...] = minv
        arg_scr[...] = argv

    @pl.when(kb == _A_BLOCKS - 1)
    def _():
        min_ref[...] = min_scr[...]
        arg_ref[...] = arg_scr[...]


def _phase_b_body(nchunk, base, x_ref, cb_hbm, amin_ref, aarg_ref,
                  idx_ref, dist_ref, cbuf, csem):
    def cpy(i, b):
        return pltpu.make_async_copy(
            cb_hbm.at[pl.ds(i * _RBLK, _RBLK)], cbuf.at[b], csem.at[b])

    for b in range(min(_NBUF, nchunk)):
        cpy(b, b).start()

    x = x_ref[...]
    x2 = jnp.sum(x * x, axis=1, keepdims=True)

    minv = amin_ref[...]
    argv = aarg_ref[...]
    for i in range(nchunk):
        b = i % _NBUF
        cpy(i, b).wait()
        c = cbuf[b]
        bmin, barg = _chunk_minarg(x, x2, c, base + i * _RBLK)
        if i + _NBUF < nchunk:
            cpy(i + _NBUF, b).start()
        minv, argv = _merge(minv, argv, bmin, barg)

    idx_ref[...] = argv
    dist_ref[...] = jnp.sqrt(minv)


def _nearest(lat_flat, cb_flat):
    b, d = lat_flat.shape
    k = cb_flat.shape[0]
    ka = _A_BLOCKS * _BLK_K
    small = pl.BlockSpec((b, 1), lambda i: (0, 0))
    amin, aarg = pl.pallas_call(
        _phase_a_body,
        grid=(_A_BLOCKS,),
        in_specs=[
            pl.BlockSpec((b, d), lambda i: (0, 0)),
            pl.BlockSpec((_BLK_K, d), lambda i: (i, 0)),
        ],
        out_specs=[small, small],
        out_shape=[
            jax.ShapeDtypeStruct((b, 1), jnp.float32),
            jax.ShapeDtypeStruct((b, 1), jnp.int32),
        ],
        scratch_shapes=[
            pltpu.VMEM((b, 1), jnp.float32),
            pltpu.VMEM((b, 1), jnp.int32),
        ],
    )(lat_flat, cb_flat)

    cb_b = lax.slice(cb_flat, (ka, 0), (k, d))
    nchunk = (k - ka) // _RBLK
    s2 = pl.BlockSpec((b, 1), lambda: (0, 0))
    idx2, dist2 = pl.pallas_call(
        functools.partial(_phase_b_body, nchunk, ka),
        in_specs=[
            pl.BlockSpec((b, d), lambda: (0, 0)),
            pl.BlockSpec(memory_space=pl.ANY),
            s2, s2,
        ],
        out_shape=[
            jax.ShapeDtypeStruct((b, 1), jnp.int32),
            jax.ShapeDtypeStruct((b, 1), jnp.float32),
        ],
        scratch_shapes=[
            pltpu.VMEM((_NBUF, _RBLK, d), jnp.float32),
            pltpu.SemaphoreType.DMA((_NBUF,)),
        ],
    )(lat_flat, cb_b, amin, aarg)
    return idx2, dist2


def _sc_gather(table, idx):
    nb = idx.shape[0]
    d = table.shape[1]
    nw = 32
    rpw = nb // nw
    mesh = plsc.VectorSubcoreMesh(core_axis_name="c", subcore_axis_name="s")

    @functools.partial(
        pl.kernel, mesh=mesh,
        out_type=jax.ShapeDtypeStruct((nb, d), jnp.float32),
        scratch_types=[
            pltpu.VMEM((nw, rpw), jnp.int32),
            pltpu.VMEM((rpw, d), jnp.float32),
            pltpu.SemaphoreType.DMA,
        ],
    )
    def k(table_hbm, idx_hbm, out_hbm, idx_v, rows_v, sem):
        wid = lax.axis_index("s") * 2 + lax.axis_index("c")
        base = wid * rpw
        pltpu.sync_copy(idx_hbm, idx_v)
        pltpu.async_copy(table_hbm.at[idx_v.at[wid]], rows_v, sem).wait()
        pltpu.sync_copy(rows_v, out_hbm.at[pl.ds(base, rpw)])

    return k(table, idx.reshape(nw, rpw))


def kernel(latents, codebook):
    latents = latents.astype(jnp.float32)
    codebook = codebook.astype(jnp.float32)
    b = latents.shape[0]
    k = codebook.shape[0]
    lat_flat = latents.reshape(b, -1)
    cb_flat = codebook.reshape(k, -1)
    idx2, dist2 = _nearest(lat_flat, cb_flat)
    indices = idx2.reshape(b)
    min_distances = dist2.reshape(b)
    quant = _sc_gather(cb_flat, indices)
    quantized = quant.reshape(latents.shape)
    return indices, quantized, min_distances


# speedup vs baseline: 1.2950x; 1.2950x over previous
import functools

import jax
import jax.numpy as jnp
from jax import lax
from jax.experimental import pallas as pl
from jax.experimental.pallas import tpu as pltpu
from jax.experimental.pallas import tpu_sc as plsc

_BLK_K = 128


def _dist_body(nkb, x_ref, c_ref, idx_ref, dist_ref, min_scr, arg_scr):
    kb = pl.program_id(0)
    x = x_ref[...]
    c = c_ref[...]
    dot = lax.dot_general(x, c, (((1,), (1,)), ((), ())),
                          preferred_element_type=jnp.float32)
    c2 = jnp.sum(c * c, axis=1)[None, :]
    x2 = jnp.sum(x * x, axis=1, keepdims=True)
    d2 = jnp.maximum(x2 + c2 - 2.0 * dot, 0.0)
    bmin = jnp.min(d2, axis=1, keepdims=True)
    col = lax.broadcasted_iota(jnp.int32, d2.shape, 1)
    barg = (jnp.min(jnp.where(d2 == bmin, col, _BLK_K), axis=1, keepdims=True)
            + kb * _BLK_K)

    @pl.when(kb == 0)
    def _():
        min_scr[...] = bmin
        arg_scr[...] = barg

    @pl.when(kb > 0)
    def _():
        better = bmin < min_scr[...]
        min_scr[...] = jnp.where(better, bmin, min_scr[...])
        arg_scr[...] = jnp.where(better, barg, arg_scr[...])

    @pl.when(kb == nkb - 1)
    def _():
        idx_ref[...] = arg_scr[...]
        dist_ref[...] = jnp.sqrt(min_scr[...])


def _nearest(lat_flat, cb_flat, interpret=False):
    b, d = lat_flat.shape
    k = cb_flat.shape[0]
    nkb = k // _BLK_K
    return pl.pallas_call(
        functools.partial(_dist_body, nkb),
        grid=(nkb,),
        in_specs=[
            pl.BlockSpec((b, d), lambda i: (0, 0)),
            pl.BlockSpec((_BLK_K, d), lambda i: (i, 0)),
        ],
        out_specs=[
            pl.BlockSpec((b, 1), lambda i: (0, 0)),
            pl.BlockSpec((b, 1), lambda i: (0, 0)),
        ],
        out_shape=[
            jax.ShapeDtypeStruct((b, 1), jnp.int32),
            jax.ShapeDtypeStruct((b, 1), jnp.float32),
        ],
        scratch_shapes=[
            pltpu.VMEM((b, 1), jnp.float32),
            pltpu.VMEM((b, 1), jnp.int32),
        ],
        interpret=interpret,
    )(lat_flat, cb_flat)


def _sc_gather(table, idx):
    nb = idx.shape[0]
    d = table.shape[1]
    nw = 32
    rpw = nb // nw
    mesh = plsc.VectorSubcoreMesh(core_axis_name="c", subcore_axis_name="s")

    @functools.partial(
        pl.kernel, mesh=mesh,
        out_type=jax.ShapeDtypeStruct((nb, d), jnp.float32),
        scratch_types=[
            pltpu.VMEM((nw, rpw), jnp.int32),
            pltpu.VMEM((rpw, d), jnp.float32),
            pltpu.SemaphoreType.DMA,
        ],
    )
    def k(table_hbm, idx_hbm, out_hbm, idx_v, rows_v, sem):
        wid = lax.axis_index("s") * 2 + lax.axis_index("c")
        base = wid * rpw
        pltpu.sync_copy(idx_hbm, idx_v)
        pltpu.async_copy(table_hbm.at[idx_v.at[wid]], rows_v, sem).wait()
        pltpu.sync_copy(rows_v, out_hbm.at[pl.ds(base, rpw)])

    return k(table, idx.reshape(nw, rpw))


def kernel(latents, codebook):
    latents = latents.astype(jnp.float32)
    codebook = codebook.astype(jnp.float32)
    b = latents.shape[0]
    k = codebook.shape[0]
    lat_flat = latents.reshape(b, -1)
    cb_flat = codebook.reshape(k, -1)
    idx2, dist2 = _nearest(lat_flat, cb_flat)
    indices = idx2.reshape(b)
    min_distances = dist2.reshape(b)
    quant = _sc_gather(cb_flat, indices)
    quantized = quant.reshape(latents.shape)
    return indices, quantized, min_distances
